# baseline (speedup 1.0000x reference)
import functools

import jax
import jax.numpy as jnp
from jax import lax
from jax.experimental import pallas as pl
from jax.experimental.pallas import tpu as pltpu
from jax.experimental.pallas import tpu_sc as plsc

BETA = 0.25
KT = 512
SC_CHUNK = 512


def _vq_body(z_ref, em2_ref, e_ref, ones_ref, codes_ref, loss_ref,
             ppl_ref, counts_ref, acc_ref, e2_ref):
    b = pl.program_id(0)
    t = pl.program_id(1)
    nb = pl.num_programs(0)
    nt = pl.num_programs(1)

    z_blk = z_ref[0]
    em2 = em2_ref[...]

    @pl.when((b == 0) & (t == 0))
    def _precompute():
        e = e_ref[...]
        e2_ref[...] = jnp.sum(e * e, axis=1, keepdims=True)

    zem = lax.dot_general(em2, z_blk, (((1,), (0,)), ((), ())),
                          preferred_element_type=jnp.float32)
    e2 = e2_ref[...]
    z2 = jnp.sum(z_blk * z_blk, axis=0, keepdims=True)
    dist = (z2 + e2) + zem

    m = jnp.min(dist, axis=0, keepdims=True)
    rows = lax.broadcasted_iota(jnp.int32, dist.shape, 0)
    idx = jnp.min(jnp.where(dist == m, rows, jnp.int32(2**30)), axis=0)
    codes_ref[0, 0, pl.ds(t * KT, KT)] = idx

    psum = jnp.sum(m)

    oh = (rows == idx[None, :]).astype(jnp.bfloat16)
    cnt = lax.dot_general(oh, ones_ref[...], (((1,), (0,)), ((), ())),
                          preferred_element_type=jnp.float32)

    @pl.when((b == 0) & (t == 0))
    def _init():
        counts_ref[...] = cnt
        acc_ref[...] = jnp.full((8, 128), psum, jnp.float32)

    @pl.when((b > 0) | (t > 0))
    def _accum():
        counts_ref[...] = counts_ref[...] + cnt
        acc_ref[...] = acc_ref[...] + psum

    @pl.when((b == nb - 1) & (t == nt - 1))
    def _finalize():
        n_elems = nb * nt * KT
        loss_ref[...] = acc_ref[...] * ((1.0 + BETA) /
                                        (n_elems * z_blk.shape[0]))
        p = counts_ref[:, 0:1] * (1.0 / n_elems)
        ent = jnp.sum(p * jnp.log(p + 1e-10))
        ppl_ref[...] = jnp.full((8, 128), jnp.exp(-ent), jnp.float32)


def _codes_and_stats(z_e, em2, codebook, ones_bf16):
    B, C, K = z_e.shape
    N = codebook.shape[0]
    grid = (B, K // KT)
    return pl.pallas_call(
        _vq_body,
        grid=grid,
        in_specs=[
            pl.BlockSpec((1, C, KT), lambda b, t: (b, 0, t)),
            pl.BlockSpec((N, C), lambda b, t: (0, 0)),
            pl.BlockSpec((N, C), lambda b, t: (0, 0)),
            pl.BlockSpec((KT, 128), lambda b, t: (0, 0)),
        ],
        out_specs=[
            pl.BlockSpec((1, 1, K), lambda b, t: (b, 0, 0)),
            pl.BlockSpec((8, 128), lambda b, t: (0, 0)),
            pl.BlockSpec((8, 128), lambda b, t: (0, 0)),
        ],
        out_shape=[
            jax.ShapeDtypeStruct((B, 1, K), jnp.int32),
            jax.ShapeDtypeStruct((8, 128), jnp.float32),
            jax.ShapeDtypeStruct((8, 128), jnp.float32),
        ],
        scratch_shapes=[
            pltpu.VMEM((N, 128), jnp.float32),
            pltpu.VMEM((8, 128), jnp.float32),
            pltpu.VMEM((N, 1), jnp.float32),
        ],
        compiler_params=pltpu.CompilerParams(
            dimension_semantics=("arbitrary", "arbitrary")),
    )(z_e, em2, codebook, ones_bf16)


def _make_sc_gather(B, C, K, N):
    info = plsc.get_sparse_core_info()
    NC, NS, L = info.num_cores, info.num_subcores, info.num_lanes
    NW = NC * NS
    BK = B * K
    pw = BK // NW
    n_chunks = pw // SC_CHUNK
    mesh = plsc.VectorSubcoreMesh(core_axis_name="c", subcore_axis_name="s")

    @functools.partial(
        pl.kernel, mesh=mesh,
        out_type=jax.ShapeDtypeStruct((B, C, K), jnp.float32),
        scratch_types=[
            pltpu.VMEM((N * C,), jnp.float32),
            pltpu.VMEM((pw,), jnp.int32),
            pltpu.VMEM((C, SC_CHUNK), jnp.float32),
        ],
        compiler_params=pltpu.CompilerParams(needs_layout_passes=False),
    )
    def sc_gather(codes_hbm, cb_hbm, zq_hbm, cb_v, codes_v, buf_v):
        wid = lax.axis_index("s") * NC + lax.axis_index("c")
        base = wid * pw
        bb = base // K
        k0 = base % K
        pltpu.sync_copy(cb_hbm, cb_v)
        pltpu.sync_copy(codes_hbm.at[pl.ds(base, pw)], codes_v)
        for chunk in range(n_chunks):
            def body(g, _):
                idx16 = codes_v[pl.ds(chunk * SC_CHUNK + g * L, L)]
                flat = idx16 * C
                for c in range(C):
                    vals = plsc.load_gather(cb_v, [flat + c])
                    buf_v[c, pl.ds(g * L, L)] = vals
                return 0
            lax.fori_loop(0, SC_CHUNK // L, body, 0)
            pltpu.sync_copy(
                buf_v, zq_hbm.at[bb, :, pl.ds(k0 + chunk * SC_CHUNK,
                                              SC_CHUNK)])

    return sc_gather


def kernel(z_e, codebook):
    B, C, K = z_e.shape
    N = codebook.shape[0]
    em2 = -2.0 * codebook
    ones_bf16 = jnp.ones((KT, 128), jnp.bfloat16)
    codes3, lossv, pplv = _codes_and_stats(z_e, em2, codebook, ones_bf16)
    codes_flat = codes3.reshape(B * K)
    zq = _make_sc_gather(B, C, K, N)(codes_flat, codebook.reshape(N * C))
    return (zq, codes3.reshape(B, K), lossv[0, 0], pplv[0, 0])


# speedup vs baseline: 1.0942x; 1.0942x over previous
import functools

import jax
import jax.numpy as jnp
from jax import lax
from jax.experimental import pallas as pl
from jax.experimental.pallas import tpu as pltpu
from jax.experimental.pallas import tpu_sc as plsc

BETA = 0.25
KT = 512
SC_CHUNK = 256


def _vq_body(z_ref, em2_ref, e_ref, ones_ref, codes_ref, loss_ref,
             ppl_ref, counts_ref, acc_ref, e2_ref):
    b = pl.program_id(0)
    t = pl.program_id(1)
    nb = pl.num_programs(0)
    nt = pl.num_programs(1)

    z_blk = z_ref[0]
    em2 = em2_ref[...]

    @pl.when((b == 0) & (t == 0))
    def _precompute():
        e = e_ref[...]
        e2_ref[...] = jnp.sum(e * e, axis=1, keepdims=True)

    zem = lax.dot_general(em2, z_blk, (((1,), (0,)), ((), ())),
                          preferred_element_type=jnp.float32)
    e2 = e2_ref[...]
    z2 = jnp.sum(z_blk * z_blk, axis=0, keepdims=True)
    dist = (z2 + e2) + zem

    m = jnp.min(dist, axis=0, keepdims=True)
    rows = lax.broadcasted_iota(jnp.int32, dist.shape, 0)
    idx = jnp.min(jnp.where(dist == m, rows, jnp.int32(2**30)), axis=0)
    codes_ref[0, 0, pl.ds(t * KT, KT)] = idx

    psum = jnp.sum(m)

    oh = (rows == idx[None, :]).astype(jnp.bfloat16)
    cnt = lax.dot_general(oh, ones_ref[...], (((1,), (0,)), ((), ())),
                          preferred_element_type=jnp.float32)

    @pl.when((b == 0) & (t == 0))
    def _init():
        counts_ref[...] = cnt
        acc_ref[...] = jnp.full((8, 128), psum, jnp.float32)

    @pl.when((b > 0) | (t > 0))
    def _accum():
        counts_ref[...] = counts_ref[...] + cnt
        acc_ref[...] = acc_ref[...] + psum

    @pl.when((b == nb - 1) & (t == nt - 1))
    def _finalize():
        n_elems = nb * nt * KT
        loss_ref[...] = acc_ref[...] * ((1.0 + BETA) /
                                        (n_elems * z_blk.shape[0]))
        p = counts_ref[:, 0:1] * (1.0 / n_elems)
        ent = jnp.sum(p * jnp.log(p + 1e-10))
        ppl_ref[...] = jnp.full((8, 128), jnp.exp(-ent), jnp.float32)


def _codes_and_stats(z_e, em2, codebook, ones_bf16):
    B, C, K = z_e.shape
    N = codebook.shape[0]
    grid = (B, K // KT)
    return pl.pallas_call(
        _vq_body,
        grid=grid,
        in_specs=[
            pl.BlockSpec((1, C, KT), lambda b, t: (b, 0, t)),
            pl.BlockSpec((N, C), lambda b, t: (0, 0)),
            pl.BlockSpec((N, C), lambda b, t: (0, 0)),
            pl.BlockSpec((KT, 128), lambda b, t: (0, 0)),
        ],
        out_specs=[
            pl.BlockSpec((1, 1, K), lambda b, t: (b, 0, 0)),
            pl.BlockSpec((8, 128), lambda b, t: (0, 0)),
            pl.BlockSpec((8, 128), lambda b, t: (0, 0)),
        ],
        out_shape=[
            jax.ShapeDtypeStruct((B, 1, K), jnp.int32),
            jax.ShapeDtypeStruct((8, 128), jnp.float32),
            jax.ShapeDtypeStruct((8, 128), jnp.float32),
        ],
        scratch_shapes=[
            pltpu.VMEM((N, 128), jnp.float32),
            pltpu.VMEM((8, 128), jnp.float32),
            pltpu.VMEM((N, 1), jnp.float32),
        ],
        compiler_params=pltpu.CompilerParams(
            dimension_semantics=("arbitrary", "arbitrary")),
    )(z_e, em2, codebook, ones_bf16)


def _make_sc_gather(B, C, K, N):
    info = plsc.get_sparse_core_info()
    NC, NS, L = info.num_cores, info.num_subcores, info.num_lanes
    NW = NC * NS
    BK = B * K
    pw = BK // NW
    n_chunks = pw // SC_CHUNK
    mesh = plsc.VectorSubcoreMesh(core_axis_name="c", subcore_axis_name="s")

    @functools.partial(
        pl.kernel, mesh=mesh,
        out_type=jax.ShapeDtypeStruct((B, C, K), jnp.float32),
        scratch_types=[
            pltpu.VMEM((N * C,), jnp.float32),
            pltpu.VMEM((pw,), jnp.int32),
            pltpu.VMEM((C, SC_CHUNK), jnp.float32),
            pltpu.VMEM((C, SC_CHUNK), jnp.float32),
            pltpu.SemaphoreType.DMA,
            pltpu.SemaphoreType.DMA,
        ],
        compiler_params=pltpu.CompilerParams(needs_layout_passes=False),
    )
    def sc_gather(codes_hbm, cb_hbm, zq_hbm, cb_v, codes_v, buf0_v, buf1_v,
                  sem0, sem1):
        wid = lax.axis_index("s") * NC + lax.axis_index("c")
        base = wid * pw
        bb = base // K
        k0 = base % K
        pltpu.sync_copy(cb_hbm, cb_v)
        pltpu.sync_copy(codes_hbm.at[pl.ds(base, pw)], codes_v)
        bufs = (buf0_v, buf1_v)
        sems = (sem0, sem1)
        handles = [None] * n_chunks
        for chunk in range(n_chunks):
            buf = bufs[chunk % 2]
            sem = sems[chunk % 2]
            if chunk >= 2:
                handles[chunk - 2].wait()

            @plsc.parallel_loop(0, SC_CHUNK // L, unroll=4)
            def body(g):
                idx16 = codes_v[pl.ds(chunk * SC_CHUNK + g * L, L)]
                flat = idx16 * C
                for c in range(C):
                    vals = plsc.load_gather(cb_v, [flat + c])
                    buf[c, pl.ds(g * L, L)] = vals

            handles[chunk] = pltpu.async_copy(
                buf, zq_hbm.at[bb, :, pl.ds(k0 + chunk * SC_CHUNK,
                                            SC_CHUNK)], sem)
        handles[n_chunks - 2].wait()
        handles[n_chunks - 1].wait()

    return sc_gather


def kernel(z_e, codebook):
    B, C, K = z_e.shape
    N = codebook.shape[0]
    em2 = -2.0 * codebook
    ones_bf16 = jnp.ones((KT, 128), jnp.bfloat16)
    codes3, lossv, pplv = _codes_and_stats(z_e, em2, codebook, ones_bf16)
    codes_flat = codes3.reshape(B * K)
    zq = _make_sc_gather(B, C, K, N)(codes_flat, codebook.reshape(N * C))
    return (zq, codes3.reshape(B, K), lossv[0, 0], pplv[0, 0])


# speedup vs baseline: 1.3733x; 1.2550x over previous
import functools

import jax
import jax.numpy as jnp
from jax import lax
from jax.experimental import pallas as pl
from jax.experimental.pallas import tpu as pltpu
from jax.experimental.pallas import tpu_sc as plsc

BETA = 0.25
KT = 512
SC_CHUNK = 256


def _vq_body(z_ref, em2_ref, e_ref, ones_ref, codes_ref, loss_ref,
             ppl_ref, counts_ref, acc_ref, e2_ref):
    b = pl.program_id(0)
    t = pl.program_id(1)
    nb = pl.num_programs(0)
    nt = pl.num_programs(1)

    z_blk = z_ref[0]
    em2 = em2_ref[...]

    @pl.when((b == 0) & (t == 0))
    def _precompute():
        e = e_ref[...]
        e2_ref[...] = jnp.sum(e * e, axis=1, keepdims=True)

    zem = lax.dot_general(em2, z_blk, (((1,), (0,)), ((), ())),
                          preferred_element_type=jnp.float32)
    e2 = e2_ref[...]
    z2 = jnp.sum(z_blk * z_blk, axis=0, keepdims=True)
    dist = (z2 + e2) + zem

    m = jnp.min(dist, axis=0, keepdims=True)
    rows = lax.broadcasted_iota(jnp.int32, dist.shape, 0)
    idx = jnp.min(jnp.where(dist == m, rows, jnp.int32(2**30)), axis=0)
    codes_ref[0, 0, pl.ds(t * KT, KT)] = idx

    psum = jnp.sum(m)

    oh = (rows == idx[None, :]).astype(jnp.bfloat16)
    cnt = lax.dot_general(oh, ones_ref[...], (((1,), (0,)), ((), ())),
                          preferred_element_type=jnp.float32)

    @pl.when((b == 0) & (t == 0))
    def _init():
        counts_ref[...] = cnt
        acc_ref[...] = jnp.full((8, 128), psum, jnp.float32)

    @pl.when((b > 0) | (t > 0))
    def _accum():
        counts_ref[...] = counts_ref[...] + cnt
        acc_ref[...] = acc_ref[...] + psum

    @pl.when((b == nb - 1) & (t == nt - 1))
    def _finalize():
        n_elems = nb * nt * KT
        loss_ref[...] = acc_ref[...] * ((1.0 + BETA) /
                                        (n_elems * z_blk.shape[0]))
        p = counts_ref[:, 0:1] * (1.0 / n_elems)
        ent = jnp.sum(p * jnp.log(p + 1e-10))
        ppl_ref[...] = jnp.full((8, 128), jnp.exp(-ent), jnp.float32)


def _codes_and_stats(z_e, em2, codebook, ones_bf16):
    B, C, K = z_e.shape
    N = codebook.shape[0]
    grid = (B, K // KT)
    return pl.pallas_call(
        _vq_body,
        grid=grid,
        in_specs=[
            pl.BlockSpec((1, C, KT), lambda b, t: (b, 0, t)),
            pl.BlockSpec((N, C), lambda b, t: (0, 0)),
            pl.BlockSpec((N, C), lambda b, t: (0, 0)),
            pl.BlockSpec((KT, 128), lambda b, t: (0, 0)),
        ],
        out_specs=[
            pl.BlockSpec((1, 1, K), lambda b, t: (b, 0, 0)),
            pl.BlockSpec((8, 128), lambda b, t: (0, 0)),
            pl.BlockSpec((8, 128), lambda b, t: (0, 0)),
        ],
        out_shape=[
            jax.ShapeDtypeStruct((B, 1, K), jnp.int32),
            jax.ShapeDtypeStruct((8, 128), jnp.float32),
            jax.ShapeDtypeStruct((8, 128), jnp.float32),
        ],
        scratch_shapes=[
            pltpu.VMEM((N, 128), jnp.float32),
            pltpu.VMEM((8, 128), jnp.float32),
            pltpu.VMEM((N, 1), jnp.float32),
        ],
        compiler_params=pltpu.CompilerParams(
            dimension_semantics=("arbitrary", "arbitrary")),
    )(z_e, em2, codebook, ones_bf16)


def _make_sc_gather(B, C, K, N):
    info = plsc.get_sparse_core_info()
    NC, NS, L = info.num_cores, info.num_subcores, info.num_lanes
    NW = NC * NS
    BK = B * K
    pw = BK // NW
    n_chunks = pw // SC_CHUNK
    mesh = plsc.VectorSubcoreMesh(core_axis_name="c", subcore_axis_name="s")

    @functools.partial(
        pl.kernel, mesh=mesh,
        out_type=jax.ShapeDtypeStruct((B, C, K), jnp.float32),
        scratch_types=[
            pltpu.VMEM((N * (C + 1),), jnp.float32),
            pltpu.VMEM((pw,), jnp.int32),
            pltpu.VMEM((C, SC_CHUNK), jnp.float32),
            pltpu.VMEM((C, SC_CHUNK), jnp.float32),
            pltpu.SemaphoreType.DMA,
            pltpu.SemaphoreType.DMA,
        ],
        compiler_params=pltpu.CompilerParams(needs_layout_passes=False),
    )
    def sc_gather(codes_hbm, cb_hbm, zq_hbm, cb_v, codes_v, buf0_v, buf1_v,
                  sem0, sem1):
        wid = lax.axis_index("s") * NC + lax.axis_index("c")
        base = wid * pw
        bb = base // K
        k0 = base % K
        pltpu.sync_copy(cb_hbm, cb_v)
        pltpu.sync_copy(codes_hbm.at[pl.ds(base, pw)], codes_v)
        bufs = (buf0_v, buf1_v)
        sems = (sem0, sem1)
        handles = [None] * n_chunks
        for chunk in range(n_chunks):
            buf = bufs[chunk % 2]
            sem = sems[chunk % 2]
            if chunk >= 2:
                handles[chunk - 2].wait()

            @plsc.parallel_loop(0, SC_CHUNK // L, unroll=4)
            def body(g):
                idx16 = codes_v[pl.ds(chunk * SC_CHUNK + g * L, L)]
                flat = idx16 * (C + 1)
                for c in range(C):
                    vals = plsc.load_gather(cb_v, [flat + c])
                    buf[c, pl.ds(g * L, L)] = vals

            handles[chunk] = pltpu.async_copy(
                buf, zq_hbm.at[bb, :, pl.ds(k0 + chunk * SC_CHUNK,
                                            SC_CHUNK)], sem)
        handles[n_chunks - 2].wait()
        handles[n_chunks - 1].wait()

    return sc_gather


def kernel(z_e, codebook):
    B, C, K = z_e.shape
    N = codebook.shape[0]
    em2 = -2.0 * codebook
    ones_bf16 = jnp.ones((KT, 128), jnp.bfloat16)
    codes3, lossv, pplv = _codes_and_stats(z_e, em2, codebook, ones_bf16)
    codes_flat = codes3.reshape(B * K)
    cb_padded = jnp.pad(codebook, ((0, 0), (0, 1))).reshape(N * (C + 1))
    zq = _make_sc_gather(B, C, K, N)(codes_flat, cb_padded)
    return (zq, codes3.reshape(B, K), lossv[0, 0], pplv[0, 0])


# speedup vs baseline: 1.7555x; 1.2783x over previous
import functools

import jax
import jax.numpy as jnp
from jax import lax
from jax.experimental import pallas as pl
from jax.experimental.pallas import tpu as pltpu
from jax.experimental.pallas import tpu_sc as plsc

BETA = 0.25
KT = 512
SC_CHUNK = 256


def _vq_body(z_ref, em2_ref, e_ref, codes_ref, loss_ref, acc_ref, e2_ref):
    b = pl.program_id(0)
    t = pl.program_id(1)
    nb = pl.num_programs(0)
    nt = pl.num_programs(1)

    z_blk = z_ref[0]
    em2 = em2_ref[...]

    @pl.when((b == 0) & (t == 0))
    def _precompute():
        e = e_ref[...]
        e2_ref[...] = jnp.sum(e * e, axis=1, keepdims=True)

    zem = lax.dot_general(em2, z_blk, (((1,), (0,)), ((), ())),
                          preferred_element_type=jnp.float32)
    e2 = e2_ref[...]
    z2 = jnp.sum(z_blk * z_blk, axis=0, keepdims=True)
    dist = (z2 + e2) + zem

    m = jnp.min(dist, axis=0, keepdims=True)
    rows = lax.broadcasted_iota(jnp.int32, dist.shape, 0)
    idx = jnp.min(jnp.where(dist == m, rows, jnp.int32(2**30)), axis=0)
    codes_ref[0, 0, pl.ds(t * KT, KT)] = idx

    psum = jnp.sum(m)

    @pl.when((b == 0) & (t == 0))
    def _init():
        acc_ref[...] = jnp.full((8, 128), psum, jnp.float32)

    @pl.when((b > 0) | (t > 0))
    def _accum():
        acc_ref[...] = acc_ref[...] + psum

    @pl.when((b == nb - 1) & (t == nt - 1))
    def _finalize():
        n_elems = nb * nt * KT
        loss_ref[...] = acc_ref[...] * ((1.0 + BETA) /
                                        (n_elems * z_blk.shape[0]))


def _codes_and_loss(z_e, em2, codebook):
    B, C, K = z_e.shape
    N = codebook.shape[0]
    grid = (B, K // KT)
    return pl.pallas_call(
        _vq_body,
        grid=grid,
        in_specs=[
            pl.BlockSpec((1, C, KT), lambda b, t: (b, 0, t)),
            pl.BlockSpec((N, C), lambda b, t: (0, 0)),
            pl.BlockSpec((N, C), lambda b, t: (0, 0)),
        ],
        out_specs=[
            pl.BlockSpec((1, 1, K), lambda b, t: (b, 0, 0)),
            pl.BlockSpec((8, 128), lambda b, t: (0, 0)),
        ],
        out_shape=[
            jax.ShapeDtypeStruct((B, 1, K), jnp.int32),
            jax.ShapeDtypeStruct((8, 128), jnp.float32),
        ],
        scratch_shapes=[
            pltpu.VMEM((8, 128), jnp.float32),
            pltpu.VMEM((N, 1), jnp.float32),
        ],
        compiler_params=pltpu.CompilerParams(
            dimension_semantics=("arbitrary", "arbitrary")),
    )(z_e, em2, codebook)


def _make_sc_gather(B, C, K, N):
    info = plsc.get_sparse_core_info()
    NC, NS, L = info.num_cores, info.num_subcores, info.num_lanes
    NW = NC * NS
    BK = B * K
    pw = BK // NW
    n_chunks = pw // SC_CHUNK
    mesh = plsc.VectorSubcoreMesh(core_axis_name="c", subcore_axis_name="s")

    @functools.partial(
        pl.kernel, mesh=mesh,
        out_type=[
            jax.ShapeDtypeStruct((B, C, K), jnp.float32),
            jax.ShapeDtypeStruct((NW, N), jnp.float32),
        ],
        scratch_types=[
            pltpu.VMEM((N * (C + 1),), jnp.float32),
            pltpu.VMEM((pw,), jnp.int32),
            pltpu.VMEM((C, SC_CHUNK), jnp.float32),
            pltpu.VMEM((C, SC_CHUNK), jnp.float32),
            pltpu.VMEM((N,), jnp.float32),
            pltpu.SemaphoreType.DMA,
            pltpu.SemaphoreType.DMA,
        ],
        compiler_params=pltpu.CompilerParams(needs_layout_passes=False),
    )
    def sc_gather(codes_hbm, cb_hbm, zq_hbm, hist_hbm, cb_v, codes_v,
                  buf0_v, buf1_v, hist_v, sem0, sem1):
        wid = lax.axis_index("s") * NC + lax.axis_index("c")
        base = wid * pw
        bb = base // K
        k0 = base % K
        pltpu.sync_copy(cb_hbm, cb_v)
        pltpu.sync_copy(codes_hbm.at[pl.ds(base, pw)], codes_v)

        @plsc.parallel_loop(0, N // L, unroll=4)
        def _zero(i):
            hist_v[pl.ds(i * L, L)] = jnp.zeros((L,), jnp.float32)

        ones16 = jnp.ones((L,), jnp.float32)
        bufs = (buf0_v, buf1_v)
        sems = (sem0, sem1)
        handles = [None] * n_chunks
        for chunk in range(n_chunks):
            buf = bufs[chunk % 2]
            sem = sems[chunk % 2]
            if chunk >= 2:
                handles[chunk - 2].wait()

            @plsc.parallel_loop(0, SC_CHUNK // L, unroll=4)
            def body(g):
                idx16 = codes_v[pl.ds(chunk * SC_CHUNK + g * L, L)]
                flat = idx16 * (C + 1)
                for c in range(C):
                    vals = plsc.load_gather(cb_v, [flat + c])
                    buf[c, pl.ds(g * L, L)] = vals
                plsc.addupdate_scatter(hist_v, [idx16], ones16)

            handles[chunk] = pltpu.async_copy(
                buf, zq_hbm.at[bb, :, pl.ds(k0 + chunk * SC_CHUNK,
                                            SC_CHUNK)], sem)
        handles[n_chunks - 2].wait()
        handles[n_chunks - 1].wait()
        pltpu.sync_copy(hist_v, hist_hbm.at[wid])

    return sc_gather


def _ppl_body(total_positions, hist_ref, ppl_ref):
    counts = jnp.sum(hist_ref[...], axis=0, keepdims=True)
    p = counts * (1.0 / total_positions)
    ent = jnp.sum(p * jnp.log(p + 1e-10))
    ppl_ref[...] = jnp.full((8, 128), jnp.exp(-ent), jnp.float32)


def _perplexity(hist, total_positions):
    NW, N = hist.shape
    return pl.pallas_call(
        functools.partial(_ppl_body, total_positions),
        in_specs=[pl.BlockSpec((NW, N), lambda: (0, 0))],
        out_specs=pl.BlockSpec((8, 128), lambda: (0, 0)),
        out_shape=jax.ShapeDtypeStruct((8, 128), jnp.float32),
    )(hist)


def kernel(z_e, codebook):
    B, C, K = z_e.shape
    N = codebook.shape[0]
    em2 = -2.0 * codebook
    codes3, lossv = _codes_and_loss(z_e, em2, codebook)
    codes_flat = codes3.reshape(B * K)
    cb_padded = jnp.pad(codebook, ((0, 0), (0, 1))).reshape(N * (C + 1))
    zq, hist = _make_sc_gather(B, C, K, N)(codes_flat, cb_padded)
    pplv = _perplexity(hist, B * K)
    return (zq, codes3.reshape(B, K), lossv[0, 0], pplv[0, 0])


# speedup vs baseline: 2.1109x; 1.2025x over previous
import functools

import jax
import jax.numpy as jnp
from jax import lax
from jax.experimental import pallas as pl
from jax.experimental.pallas import tpu as pltpu
from jax.experimental.pallas import tpu_sc as plsc

BETA = 0.25
KT = 1024
SC_CHUNK = 256


def _vq_body(z_ref, em2_ref, e_ref, codes_ref, loss_ref, acc_ref, e2_ref):
    b = pl.program_id(0)
    t = pl.program_id(1)
    nb = pl.num_programs(0)
    nt = pl.num_programs(1)

    z_blk = z_ref[0]
    em2 = em2_ref[...]

    @pl.when((b == 0) & (t == 0))
    def _precompute():
        e = e_ref[...]
        e2_ref[...] = jnp.sum(e * e, axis=1, keepdims=True)

    zem = lax.dot_general(em2, z_blk, (((1,), (0,)), ((), ())),
                          preferred_element_type=jnp.float32)
    e2 = e2_ref[...]
    z2 = jnp.sum(z_blk * z_blk, axis=0, keepdims=True)
    dist = (z2 + e2) + zem

    m = jnp.min(dist, axis=0, keepdims=True)
    rows = lax.broadcasted_iota(jnp.int32, dist.shape, 0)
    idx = jnp.min(jnp.where(dist == m, rows, jnp.int32(2**30)), axis=0)
    codes_ref[0, 0, pl.ds(t * KT, KT)] = idx

    psum = jnp.sum(m)

    @pl.when((b == 0) & (t == 0))
    def _init():
        acc_ref[...] = jnp.full((8, 128), psum, jnp.float32)

    @pl.when((b > 0) | (t > 0))
    def _accum():
        acc_ref[...] = acc_ref[...] + psum

    @pl.when((b == nb - 1) & (t == nt - 1))
    def _finalize():
        n_elems = nb * nt * KT
        loss_ref[...] = acc_ref[...] * ((1.0 + BETA) /
                                        (n_elems * z_blk.shape[0]))


def _codes_and_loss(z_e, em2, codebook):
    B, C, K = z_e.shape
    N = codebook.shape[0]
    grid = (B, K // KT)
    return pl.pallas_call(
        _vq_body,
        grid=grid,
        in_specs=[
            pl.BlockSpec((1, C, KT), lambda b, t: (b, 0, t)),
            pl.BlockSpec((N, C), lambda b, t: (0, 0)),
            pl.BlockSpec((N, C), lambda b, t: (0, 0)),
        ],
        out_specs=[
            pl.BlockSpec((1, 1, K), lambda b, t: (b, 0, 0)),
            pl.BlockSpec((8, 128), lambda b, t: (0, 0)),
        ],
        out_shape=[
            jax.ShapeDtypeStruct((B, 1, K), jnp.int32),
            jax.ShapeDtypeStruct((8, 128), jnp.float32),
        ],
        scratch_shapes=[
            pltpu.VMEM((8, 128), jnp.float32),
            pltpu.VMEM((N, 1), jnp.float32),
        ],
        compiler_params=pltpu.CompilerParams(
            dimension_semantics=("arbitrary", "arbitrary")),
    )(z_e, em2, codebook)


def _make_sc_gather(B, C, K, N):
    info = plsc.get_sparse_core_info()
    NC, NS, L = info.num_cores, info.num_subcores, info.num_lanes
    NW = NC * NS
    BK = B * K
    pw = BK // NW
    n_chunks = pw // SC_CHUNK
    mesh = plsc.VectorSubcoreMesh(core_axis_name="c", subcore_axis_name="s")

    @functools.partial(
        pl.kernel, mesh=mesh,
        out_type=[
            jax.ShapeDtypeStruct((B, C, K), jnp.float32),
            jax.ShapeDtypeStruct((NW, N), jnp.float32),
        ],
        scratch_types=[
            pltpu.VMEM((N * (C + 1),), jnp.float32),
            pltpu.VMEM((pw,), jnp.int32),
            pltpu.VMEM((C, SC_CHUNK), jnp.float32),
            pltpu.VMEM((C, SC_CHUNK), jnp.float32),
            pltpu.VMEM((N,), jnp.float32),
            pltpu.SemaphoreType.DMA,
            pltpu.SemaphoreType.DMA,
        ],
        compiler_params=pltpu.CompilerParams(needs_layout_passes=False),
    )
    def sc_gather(codes_hbm, cb_hbm, zq_hbm, hist_hbm, cb_v, codes_v,
                  buf0_v, buf1_v, hist_v, sem0, sem1):
        wid = lax.axis_index("s") * NC + lax.axis_index("c")
        base = wid * pw
        bb = base // K
        k0 = base % K
        pltpu.sync_copy(cb_hbm, cb_v)
        pltpu.sync_copy(codes_hbm.at[pl.ds(base, pw)], codes_v)

        @plsc.parallel_loop(0, N // L, unroll=4)
        def _zero(i):
            hist_v[pl.ds(i * L, L)] = jnp.zeros((L,), jnp.float32)

        ones16 = jnp.ones((L,), jnp.float32)
        bufs = (buf0_v, buf1_v)
        sems = (sem0, sem1)
        handles = [None] * n_chunks
        for chunk in range(n_chunks):
            buf = bufs[chunk % 2]
            sem = sems[chunk % 2]
            if chunk >= 2:
                handles[chunk - 2].wait()

            @plsc.parallel_loop(0, SC_CHUNK // L, unroll=4)
            def body(g):
                idx16 = codes_v[pl.ds(chunk * SC_CHUNK + g * L, L)]
                flat = idx16 * (C + 1)
                for c in range(C):
                    vals = plsc.load_gather(cb_v, [flat + c])
                    buf[c, pl.ds(g * L, L)] = vals
                plsc.addupdate_scatter(hist_v, [idx16], ones16)

            handles[chunk] = pltpu.async_copy(
                buf, zq_hbm.at[bb, :, pl.ds(k0 + chunk * SC_CHUNK,
                                            SC_CHUNK)], sem)
        handles[n_chunks - 2].wait()
        handles[n_chunks - 1].wait()
        pltpu.sync_copy(hist_v, hist_hbm.at[wid])

    return sc_gather


def _ppl_body(total_positions, hist_ref, ppl_ref):
    counts = jnp.sum(hist_ref[...], axis=0, keepdims=True)
    p = counts * (1.0 / total_positions)
    ent = jnp.sum(p * jnp.log(p + 1e-10))
    ppl_ref[...] = jnp.full((8, 128), jnp.exp(-ent), jnp.float32)


def _perplexity(hist, total_positions):
    NW, N = hist.shape
    return pl.pallas_call(
        functools.partial(_ppl_body, total_positions),
        in_specs=[pl.BlockSpec((NW, N), lambda: (0, 0))],
        out_specs=pl.BlockSpec((8, 128), lambda: (0, 0)),
        out_shape=jax.ShapeDtypeStruct((8, 128), jnp.float32),
    )(hist)


def kernel(z_e, codebook):
    B, C, K = z_e.shape
    N = codebook.shape[0]
    em2 = -2.0 * codebook
    codes3, lossv = _codes_and_loss(z_e, em2, codebook)
    codes_flat = codes3.reshape(B * K)
    cb_padded = jnp.pad(codebook, ((0, 0), (0, 1))).reshape(N * (C + 1))
    zq, hist = _make_sc_gather(B, C, K, N)(codes_flat, cb_padded)
    pplv = _perplexity(hist, B * K)
    return (zq, codes3.reshape(B, K), lossv[0, 0], pplv[0, 0])


# speedup vs baseline: 2.2431x; 1.0627x over previous
import functools

import jax
import jax.numpy as jnp
from jax import lax
from jax.experimental import pallas as pl
from jax.experimental.pallas import tpu as pltpu
from jax.experimental.pallas import tpu_sc as plsc

BETA = 0.25
KT = 2048
SC_CHUNK = 256


def _vq_body(z_ref, em2_ref, e_ref, codes_ref, loss_ref, acc_ref, e2_ref):
    b = pl.program_id(0)
    t = pl.program_id(1)
    nb = pl.num_programs(0)
    nt = pl.num_programs(1)

    z_blk = z_ref[0]
    em2 = em2_ref[...]

    @pl.when((b == 0) & (t == 0))
    def _precompute():
        e = e_ref[...]
        e2_ref[...] = jnp.sum(e * e, axis=1, keepdims=True)

    zem = lax.dot_general(em2, z_blk, (((1,), (0,)), ((), ())),
                          preferred_element_type=jnp.float32)
    e2 = e2_ref[...]
    z2 = jnp.sum(z_blk * z_blk, axis=0, keepdims=True)
    dist = (z2 + e2) + zem

    m = jnp.min(dist, axis=0, keepdims=True)
    rows = lax.broadcasted_iota(jnp.int32, dist.shape, 0)
    idx = jnp.min(jnp.where(dist == m, rows, jnp.int32(2**30)), axis=0)
    codes_ref[0, 0, pl.ds(t * KT, KT)] = idx

    psum = jnp.sum(m)

    @pl.when((b == 0) & (t == 0))
    def _init():
        acc_ref[...] = jnp.full((8, 128), psum, jnp.float32)

    @pl.when((b > 0) | (t > 0))
    def _accum():
        acc_ref[...] = acc_ref[...] + psum

    @pl.when((b == nb - 1) & (t == nt - 1))
    def _finalize():
        n_elems = nb * nt * KT
        loss_ref[...] = acc_ref[...] * ((1.0 + BETA) /
                                        (n_elems * z_blk.shape[0]))


def _codes_and_loss(z_e, em2, codebook):
    B, C, K = z_e.shape
    N = codebook.shape[0]
    grid = (B, K // KT)
    return pl.pallas_call(
        _vq_body,
        grid=grid,
        in_specs=[
            pl.BlockSpec((1, C, KT), lambda b, t: (b, 0, t)),
            pl.BlockSpec((N, C), lambda b, t: (0, 0)),
            pl.BlockSpec((N, C), lambda b, t: (0, 0)),
        ],
        out_specs=[
            pl.BlockSpec((1, 1, K), lambda b, t: (b, 0, 0)),
            pl.BlockSpec((8, 128), lambda b, t: (0, 0)),
        ],
        out_shape=[
            jax.ShapeDtypeStruct((B, 1, K), jnp.int32),
            jax.ShapeDtypeStruct((8, 128), jnp.float32),
        ],
        scratch_shapes=[
            pltpu.VMEM((8, 128), jnp.float32),
            pltpu.VMEM((N, 1), jnp.float32),
        ],
        compiler_params=pltpu.CompilerParams(
            dimension_semantics=("arbitrary", "arbitrary")),
    )(z_e, em2, codebook)


def _make_sc_gather(B, C, K, N):
    info = plsc.get_sparse_core_info()
    NC, NS, L = info.num_cores, info.num_subcores, info.num_lanes
    NW = NC * NS
    BK = B * K
    pw = BK // NW
    n_chunks = pw // SC_CHUNK
    mesh = plsc.VectorSubcoreMesh(core_axis_name="c", subcore_axis_name="s")

    @functools.partial(
        pl.kernel, mesh=mesh,
        out_type=[
            jax.ShapeDtypeStruct((B, C, K), jnp.float32),
            jax.ShapeDtypeStruct((NW, N), jnp.float32),
        ],
        scratch_types=[
            pltpu.VMEM((N * (C + 1),), jnp.float32),
            pltpu.VMEM((pw,), jnp.int32),
            pltpu.VMEM((C, SC_CHUNK), jnp.float32),
            pltpu.VMEM((C, SC_CHUNK), jnp.float32),
            pltpu.VMEM((N,), jnp.float32),
            pltpu.SemaphoreType.DMA,
            pltpu.SemaphoreType.DMA,
        ],
        compiler_params=pltpu.CompilerParams(needs_layout_passes=False),
    )
    def sc_gather(codes_hbm, cb_hbm, zq_hbm, hist_hbm, cb_v, codes_v,
                  buf0_v, buf1_v, hist_v, sem0, sem1):
        wid = lax.axis_index("s") * NC + lax.axis_index("c")
        base = wid * pw
        bb = base // K
        k0 = base % K
        pltpu.sync_copy(cb_hbm, cb_v)
        pltpu.sync_copy(codes_hbm.at[pl.ds(base, pw)], codes_v)

        @plsc.parallel_loop(0, N // L, unroll=4)
        def _zero(i):
            hist_v[pl.ds(i * L, L)] = jnp.zeros((L,), jnp.float32)

        ones16 = jnp.ones((L,), jnp.float32)
        bufs = (buf0_v, buf1_v)
        sems = (sem0, sem1)
        handles = [None] * n_chunks
        for chunk in range(n_chunks):
            buf = bufs[chunk % 2]
            sem = sems[chunk % 2]
            if chunk >= 2:
                handles[chunk - 2].wait()

            @plsc.parallel_loop(0, SC_CHUNK // L, unroll=4)
            def body(g):
                idx16 = codes_v[pl.ds(chunk * SC_CHUNK + g * L, L)]
                flat = idx16 * (C + 1)
                for c in range(C):
                    vals = plsc.load_gather(cb_v, [flat + c])
                    buf[c, pl.ds(g * L, L)] = vals
                plsc.addupdate_scatter(hist_v, [idx16], ones16)

            handles[chunk] = pltpu.async_copy(
                buf, zq_hbm.at[bb, :, pl.ds(k0 + chunk * SC_CHUNK,
                                            SC_CHUNK)], sem)
        handles[n_chunks - 2].wait()
        handles[n_chunks - 1].wait()
        pltpu.sync_copy(hist_v, hist_hbm.at[wid])

    return sc_gather


def _ppl_body(total_positions, hist_ref, ppl_ref):
    counts = jnp.sum(hist_ref[...], axis=0, keepdims=True)
    p = counts * (1.0 / total_positions)
    ent = jnp.sum(p * jnp.log(p + 1e-10))
    ppl_ref[...] = jnp.full((8, 128), jnp.exp(-ent), jnp.float32)


def _perplexity(hist, total_positions):
    NW, N = hist.shape
    return pl.pallas_call(
        functools.partial(_ppl_body, total_positions),
        in_specs=[pl.BlockSpec((NW, N), lambda: (0, 0))],
        out_specs=pl.BlockSpec((8, 128), lambda: (0, 0)),
        out_shape=jax.ShapeDtypeStruct((8, 128), jnp.float32),
    )(hist)


def kernel(z_e, codebook):
    B, C, K = z_e.shape
    N = codebook.shape[0]
    em2 = -2.0 * codebook
    codes3, lossv = _codes_and_loss(z_e, em2, codebook)
    codes_flat = codes3.reshape(B * K)
    cb_padded = jnp.pad(codebook, ((0, 0), (0, 1))).reshape(N * (C + 1))
    zq, hist = _make_sc_gather(B, C, K, N)(codes_flat, cb_padded)
    pplv = _perplexity(hist, B * K)
    return (zq, codes3.reshape(B, K), lossv[0, 0], pplv[0, 0])


# speedup vs baseline: 2.3395x; 1.0430x over previous
import functools

import jax
import jax.numpy as jnp
from jax import lax
from jax.experimental import pallas as pl
from jax.experimental.pallas import tpu as pltpu
from jax.experimental.pallas import tpu_sc as plsc

BETA = 0.25
KT = 4096
SC_CHUNK = 256


def _vq_body(z_ref, em2_ref, e_ref, codes_ref, loss_ref, acc_ref, e2_ref):
    b = pl.program_id(0)
    t = pl.program_id(1)
    nb = pl.num_programs(0)
    nt = pl.num_programs(1)

    z_blk = z_ref[0]
    em2 = em2_ref[...]

    @pl.when((b == 0) & (t == 0))
    def _precompute():
        e = e_ref[...]
        e2_ref[...] = jnp.sum(e * e, axis=1, keepdims=True)

    zem = lax.dot_general(em2, z_blk, (((1,), (0,)), ((), ())),
                          preferred_element_type=jnp.float32)
    e2 = e2_ref[...]
    z2 = jnp.sum(z_blk * z_blk, axis=0, keepdims=True)
    dist = (z2 + e2) + zem

    m = jnp.min(dist, axis=0, keepdims=True)
    rows = lax.broadcasted_iota(jnp.int32, dist.shape, 0)
    idx = jnp.min(jnp.where(dist == m, rows, jnp.int32(2**30)), axis=0)
    codes_ref[0, 0, pl.ds(t * KT, KT)] = idx

    psum = jnp.sum(m)

    @pl.when((b == 0) & (t == 0))
    def _init():
        acc_ref[...] = jnp.full((8, 128), psum, jnp.float32)

    @pl.when((b > 0) | (t > 0))
    def _accum():
        acc_ref[...] = acc_ref[...] + psum

    @pl.when((b == nb - 1) & (t == nt - 1))
    def _finalize():
        n_elems = nb * nt * KT
        loss_ref[...] = acc_ref[...] * ((1.0 + BETA) /
                                        (n_elems * z_blk.shape[0]))


def _codes_and_loss(z_e, em2, codebook):
    B, C, K = z_e.shape
    N = codebook.shape[0]
    grid = (B, K // KT)
    return pl.pallas_call(
        _vq_body,
        grid=grid,
        in_specs=[
            pl.BlockSpec((1, C, KT), lambda b, t: (b, 0, t)),
            pl.BlockSpec((N, C), lambda b, t: (0, 0)),
            pl.BlockSpec((N, C), lambda b, t: (0, 0)),
        ],
        out_specs=[
            pl.BlockSpec((1, 1, K), lambda b, t: (b, 0, 0)),
            pl.BlockSpec((8, 128), lambda b, t: (0, 0)),
        ],
        out_shape=[
            jax.ShapeDtypeStruct((B, 1, K), jnp.int32),
            jax.ShapeDtypeStruct((8, 128), jnp.float32),
        ],
        scratch_shapes=[
            pltpu.VMEM((8, 128), jnp.float32),
            pltpu.VMEM((N, 1), jnp.float32),
        ],
        compiler_params=pltpu.CompilerParams(
            dimension_semantics=("arbitrary", "arbitrary")),
    )(z_e, em2, codebook)


def _make_sc_gather(B, C, K, N):
    info = plsc.get_sparse_core_info()
    NC, NS, L = info.num_cores, info.num_subcores, info.num_lanes
    NW = NC * NS
    BK = B * K
    pw = BK // NW
    n_chunks = pw // SC_CHUNK
    mesh = plsc.VectorSubcoreMesh(core_axis_name="c", subcore_axis_name="s")

    @functools.partial(
        pl.kernel, mesh=mesh,
        out_type=[
            jax.ShapeDtypeStruct((B, C, K), jnp.float32),
            jax.ShapeDtypeStruct((NW, N), jnp.float32),
        ],
        scratch_types=[
            pltpu.VMEM((N * (C + 1),), jnp.float32),
            pltpu.VMEM((pw,), jnp.int32),
            pltpu.VMEM((C, SC_CHUNK), jnp.float32),
            pltpu.VMEM((C, SC_CHUNK), jnp.float32),
            pltpu.VMEM((N,), jnp.float32),
            pltpu.SemaphoreType.DMA,
            pltpu.SemaphoreType.DMA,
        ],
        compiler_params=pltpu.CompilerParams(needs_layout_passes=False),
    )
    def sc_gather(codes_hbm, cb_hbm, zq_hbm, hist_hbm, cb_v, codes_v,
                  buf0_v, buf1_v, hist_v, sem0, sem1):
        wid = lax.axis_index("s") * NC + lax.axis_index("c")
        base = wid * pw
        bb = base // K
        k0 = base % K
        pltpu.sync_copy(cb_hbm, cb_v)
        pltpu.sync_copy(codes_hbm.at[pl.ds(base, pw)], codes_v)

        @plsc.parallel_loop(0, N // L, unroll=4)
        def _zero(i):
            hist_v[pl.ds(i * L, L)] = jnp.zeros((L,), jnp.float32)

        ones16 = jnp.ones((L,), jnp.float32)
        bufs = (buf0_v, buf1_v)
        sems = (sem0, sem1)
        handles = [None] * n_chunks
        for chunk in range(n_chunks):
            buf = bufs[chunk % 2]
            sem = sems[chunk % 2]
            if chunk >= 2:
                handles[chunk - 2].wait()

            @plsc.parallel_loop(0, SC_CHUNK // L, unroll=4)
            def body(g):
                idx16 = codes_v[pl.ds(chunk * SC_CHUNK + g * L, L)]
                flat = idx16 * (C + 1)
                for c in range(C):
                    vals = plsc.load_gather(cb_v, [flat + c])
                    buf[c, pl.ds(g * L, L)] = vals
                plsc.addupdate_scatter(hist_v, [idx16], ones16)

            handles[chunk] = pltpu.async_copy(
                buf, zq_hbm.at[bb, :, pl.ds(k0 + chunk * SC_CHUNK,
                                            SC_CHUNK)], sem)
        handles[n_chunks - 2].wait()
        handles[n_chunks - 1].wait()
        pltpu.sync_copy(hist_v, hist_hbm.at[wid])

    return sc_gather


def _ppl_body(total_positions, hist_ref, ppl_ref):
    counts = jnp.sum(hist_ref[...], axis=0, keepdims=True)
    p = counts * (1.0 / total_positions)
    ent = jnp.sum(p * jnp.log(p + 1e-10))
    ppl_ref[...] = jnp.full((8, 128), jnp.exp(-ent), jnp.float32)


def _perplexity(hist, total_positions):
    NW, N = hist.shape
    return pl.pallas_call(
        functools.partial(_ppl_body, total_positions),
        in_specs=[pl.BlockSpec((NW, N), lambda: (0, 0))],
        out_specs=pl.BlockSpec((8, 128), lambda: (0, 0)),
        out_shape=jax.ShapeDtypeStruct((8, 128), jnp.float32),
    )(hist)


def kernel(z_e, codebook):
    B, C, K = z_e.shape
    N = codebook.shape[0]
    em2 = -2.0 * codebook
    codes3, lossv = _codes_and_loss(z_e, em2, codebook)
    codes_flat = codes3.reshape(B * K)
    cb_padded = jnp.pad(codebook, ((0, 0), (0, 1))).reshape(N * (C + 1))
    zq, hist = _make_sc_gather(B, C, K, N)(codes_flat, cb_padded)
    pplv = _perplexity(hist, B * K)
    return (zq, codes3.reshape(B, K), lossv[0, 0], pplv[0, 0])
